# Initial kernel scaffold; baseline (speedup 1.0000x reference)
import jax
import jax.numpy as jnp
from jax.experimental import pallas as pl


def kernel(tokens, emb, W, b):
    raise NotImplementedError("write your pallas kernel here")


# speedup vs baseline: 2.5412x; 2.5412x over previous
import functools

import jax
import jax.numpy as jnp
from jax import lax
from jax.experimental import pallas as pl
from jax.experimental.pallas import tpu as pltpu
from jax.experimental.pallas import tpu_sc as plsc

_LANES = 16


def _make_sc_pool(B, L, V, D, num_cores, num_subcores):
    NW = num_cores * num_subcores
    assert B % NW == 0
    b_per_w = B // NW
    D16 = D // _LANES
    CH = 40
    assert L % CH == 0 and CH % 8 == 0
    n_chunks = L // CH

    mesh = plsc.VectorSubcoreMesh(core_axis_name="c", subcore_axis_name="s")

    @functools.partial(
        pl.kernel,
        mesh=mesh,
        out_type=jax.ShapeDtypeStruct((B, D), jnp.float32),
        scratch_types=[
            pltpu.VMEM((b_per_w * L,), jnp.int32),
            pltpu.VMEM((CH, D), jnp.float32),
            pltpu.VMEM((D,), jnp.float32),
            pltpu.SemaphoreType.DMA,
        ],
    )
    def sc_pool(tok_hbm, table_hbm, out_hbm, tok_v, rows_v, acc_v, sem):
        wid = lax.axis_index("s") * num_cores + lax.axis_index("c")
        base = wid * b_per_w
        pltpu.sync_copy(tok_hbm.at[pl.ds(base * L, b_per_w * L)], tok_v)

        def batch_body(i, carry):
            def chunk_body(c, accs):
                pltpu.async_copy(
                    table_hbm.at[tok_v.at[pl.ds(i * L + c * CH, CH)]], rows_v, sem
                ).wait()

                def row_body(r, accs):
                    return tuple(
                        accs[j] + rows_v[r, pl.ds(j * _LANES, _LANES)]
                        for j in range(D16)
                    )

                return lax.fori_loop(0, CH, row_body, accs)

            zeros = tuple(jnp.zeros((_LANES,), jnp.float32) for _ in range(D16))
            accs = lax.fori_loop(0, n_chunks, chunk_body, zeros)
            for j in range(D16):
                acc_v[pl.ds(j * _LANES, _LANES)] = accs[j]
            pltpu.sync_copy(acc_v, out_hbm.at[base + i])
            return carry

        lax.fori_loop(0, b_per_w, batch_body, 0)

    return sc_pool


def _tc_proj_body(x_ref, w_ref, b_ref, o_ref, *, inv_l):
    x = x_ref[...] * inv_l
    y = lax.dot_general(
        x, w_ref[...], (((1,), (1,)), ((), ())),
        preferred_element_type=jnp.float32,
    )
    y = y + b_ref[...]
    norm = jnp.sqrt(jnp.sum(y * y, axis=-1, keepdims=True))
    o_ref[...] = y / jnp.maximum(norm, 1e-12)


def _tc_proj(x, W, b2d, L):
    B, D = x.shape
    BB = 256
    assert B % BB == 0
    return pl.pallas_call(
        functools.partial(_tc_proj_body, inv_l=1.0 / L),
        grid=(B // BB,),
        in_specs=[
            pl.BlockSpec((BB, D), lambda i: (i, 0)),
            pl.BlockSpec((D, D), lambda i: (0, 0)),
            pl.BlockSpec((1, D), lambda i: (0, 0)),
        ],
        out_specs=pl.BlockSpec((BB, D), lambda i: (i, 0)),
        out_shape=jax.ShapeDtypeStruct((B, D), jnp.float32),
    )(x, W, b2d)


def kernel(tokens, emb, W, b):
    B, L = tokens.shape
    V, D = emb.shape
    info = plsc.get_sparse_core_info()
    sc_pool = _make_sc_pool(B, L, V, D, info.num_cores, info.num_subcores)
    pooled = sc_pool(tokens.astype(jnp.int32).reshape(B * L), emb)
    return _tc_proj(pooled, W, b.reshape(1, D), L)


# speedup vs baseline: 3.9222x; 1.5434x over previous
import functools

import jax
import jax.numpy as jnp
from jax import lax
from jax.experimental import pallas as pl
from jax.experimental.pallas import tpu as pltpu
from jax.experimental.pallas import tpu_sc as plsc

_LANES = 16


def _make_sc_pool(B, L, V, D, num_cores, num_subcores):
    NW = num_cores * num_subcores
    assert B % NW == 0
    b_per_w = B // NW
    assert b_per_w % 2 == 0
    D16 = D // _LANES
    CH = 40
    assert L % CH == 0 and (CH * 4) % 8 == 0
    n_chunks = L // CH
    n_tot = b_per_w * n_chunks

    mesh = plsc.VectorSubcoreMesh(core_axis_name="c", subcore_axis_name="s")

    @functools.partial(
        pl.kernel,
        mesh=mesh,
        out_type=jax.ShapeDtypeStruct((B, D), jnp.float32),
        scratch_types=[
            pltpu.VMEM((b_per_w * L,), jnp.int32),
            pltpu.VMEM((2, CH, D), jnp.float32),
            pltpu.VMEM((2, D), jnp.float32),
            pltpu.SemaphoreType.DMA((2,)),
            pltpu.SemaphoreType.DMA((2,)),
        ],
    )
    def sc_pool(tok_hbm, table_hbm, out_hbm, tok_v, rows_v, stage_v, gsem, osem):
        wid = lax.axis_index("s") * num_cores + lax.axis_index("c")
        base = wid * b_per_w
        pltpu.sync_copy(tok_hbm.at[pl.ds(base * L, b_per_w * L)], tok_v)

        def gather(k, s):
            return pltpu.make_async_copy(
                table_hbm.at[tok_v.at[pl.ds(k * CH, CH)]],
                rows_v.at[s],
                gsem.at[s],
            )

        gather(0, 0).start()

        def chunk_body(k, carry):
            accs = carry
            s = lax.rem(k, 2)
            gather(k, s).wait()

            @pl.when(k + 1 < n_tot)
            def _():
                gather(k + 1, 1 - s).start()

            def row_body(r, accs):
                return tuple(
                    accs[j] + rows_v[s, r, pl.ds(j * _LANES, _LANES)]
                    for j in range(D16)
                )

            accs = lax.fori_loop(0, CH, row_body, accs)

            c = lax.rem(k, n_chunks)
            i = lax.div(k, n_chunks)
            q = lax.rem(i, 2)

            @pl.when(c == n_chunks - 1)
            def _():
                @pl.when(i >= 2)
                def _():
                    pltpu.make_async_copy(
                        stage_v.at[q], out_hbm.at[base + i - 2], osem.at[q]
                    ).wait()
                for j in range(D16):
                    stage_v[q, pl.ds(j * _LANES, _LANES)] = accs[j]
                pltpu.make_async_copy(
                    stage_v.at[q], out_hbm.at[base + i], osem.at[q]
                ).start()

            zeros = tuple(jnp.zeros((_LANES,), jnp.float32) for _ in range(D16))
            return tuple(
                jnp.where(c == n_chunks - 1, z, a) for z, a in zip(zeros, accs)
            )

        zeros = tuple(jnp.zeros((_LANES,), jnp.float32) for _ in range(D16))
        lax.fori_loop(0, n_tot, chunk_body, zeros)

        pltpu.make_async_copy(
            stage_v.at[0], out_hbm.at[base + b_per_w - 2], osem.at[0]
        ).wait()
        pltpu.make_async_copy(
            stage_v.at[1], out_hbm.at[base + b_per_w - 1], osem.at[1]
        ).wait()

    return sc_pool


def _tc_proj_body(x_ref, w_ref, b_ref, o_ref, *, inv_l):
    x = x_ref[...] * inv_l
    y = lax.dot_general(
        x, w_ref[...], (((1,), (1,)), ((), ())),
        preferred_element_type=jnp.float32,
    )
    y = y + b_ref[...]
    norm = jnp.sqrt(jnp.sum(y * y, axis=-1, keepdims=True))
    o_ref[...] = y / jnp.maximum(norm, 1e-12)


def _tc_proj(x, W, b2d, L):
    B, D = x.shape
    BB = 256
    assert B % BB == 0
    return pl.pallas_call(
        functools.partial(_tc_proj_body, inv_l=1.0 / L),
        grid=(B // BB,),
        in_specs=[
            pl.BlockSpec((BB, D), lambda i: (i, 0)),
            pl.BlockSpec((D, D), lambda i: (0, 0)),
            pl.BlockSpec((1, D), lambda i: (0, 0)),
        ],
        out_specs=pl.BlockSpec((BB, D), lambda i: (i, 0)),
        out_shape=jax.ShapeDtypeStruct((B, D), jnp.float32),
    )(x, W, b2d)


def kernel(tokens, emb, W, b):
    B, L = tokens.shape
    V, D = emb.shape
    info = plsc.get_sparse_core_info()
    sc_pool = _make_sc_pool(B, L, V, D, info.num_cores, info.num_subcores)
    pooled = sc_pool(tokens.astype(jnp.int32).reshape(B * L), emb)
    return _tc_proj(pooled, W, b.reshape(1, D), L)
